# baseline (speedup 1.0000x reference)
import functools
import math

import jax
import jax.numpy as jnp
from jax import lax
from jax.experimental import pallas as pl
from jax.experimental.pallas import tpu as pltpu
from jax.experimental.pallas import tpu_sc as plsc

D_MODEL = 64
SCALE = math.sqrt(D_MODEL)

NUM_CORES = 2
NUM_SUBCORES = 16
NUM_WORKERS = NUM_CORES * NUM_SUBCORES
LANES = 16

CHUNK = 512


def _emb_kernel(n_rows):
    b_per_w = n_rows // NUM_WORKERS
    n_chunks = b_per_w // CHUNK
    mesh = plsc.VectorSubcoreMesh(core_axis_name="c", subcore_axis_name="s")

    @functools.partial(
        pl.kernel,
        mesh=mesh,
        out_type=jax.ShapeDtypeStruct((n_rows, D_MODEL), jnp.float32),
        scratch_types=[
            pltpu.VMEM((CHUNK,), jnp.int32),
            pltpu.VMEM((CHUNK, D_MODEL), jnp.float32),
            pltpu.SemaphoreType.DMA,
        ],
        compiler_params=pltpu.CompilerParams(use_tc_tiling_on_sc=False),
    )
    def k(x_hbm, table_hbm, out_hbm, idx_v, rows_v, sem):
        cid = lax.axis_index("c")
        sid = lax.axis_index("s")
        wid = sid * NUM_CORES + cid
        base = wid * b_per_w

        def chunk_body(i, carry):
            off = base + i * CHUNK

            pltpu.sync_copy(x_hbm.at[pl.ds(off, CHUNK)], idx_v)
            pltpu.async_copy(table_hbm.at[idx_v], rows_v, sem).wait()

            def scale_row(r, carry2):
                for c4 in range(D_MODEL // LANES):
                    sl = pl.ds(c4 * LANES, LANES)
                    rows_v[r, sl] = rows_v[r, sl] * SCALE
                return carry2

            lax.fori_loop(0, CHUNK, scale_row, 0, unroll=4)

            pltpu.sync_copy(rows_v, out_hbm.at[pl.ds(off, CHUNK)])
            return carry

        lax.fori_loop(0, n_chunks, chunk_body, 0)

    return k


def kernel(x, table):
    b0, b1 = x.shape
    n_rows = b0 * b1
    x_flat = x.reshape(n_rows).astype(jnp.int32)
    out = _emb_kernel(n_rows)(x_flat, table)
    return out.reshape(b0, b1, D_MODEL)


# speedup vs baseline: 1.0815x; 1.0815x over previous
import functools
import math

import jax
import jax.numpy as jnp
from jax import lax
from jax.experimental import pallas as pl
from jax.experimental.pallas import tpu as pltpu
from jax.experimental.pallas import tpu_sc as plsc

D_MODEL = 64
SCALE = math.sqrt(D_MODEL)

NUM_CORES = 2
NUM_SUBCORES = 16
NUM_WORKERS = NUM_CORES * NUM_SUBCORES
LANES = 16

CHUNK = 800


def _emb_kernel(n_rows):
    b_per_w = n_rows // NUM_WORKERS
    n_chunks = b_per_w // CHUNK
    assert n_chunks * CHUNK == b_per_w and n_chunks % 2 == 0
    mesh = plsc.VectorSubcoreMesh(core_axis_name="c", subcore_axis_name="s")

    @functools.partial(
        pl.kernel,
        mesh=mesh,
        out_type=jax.ShapeDtypeStruct((n_rows, D_MODEL), jnp.float32),
        scratch_types=[
            pltpu.VMEM((b_per_w,), jnp.int32),
            pltpu.VMEM((CHUNK, D_MODEL), jnp.float32),
            pltpu.VMEM((CHUNK, D_MODEL), jnp.float32),
            pltpu.SemaphoreType.DMA,
            pltpu.SemaphoreType.DMA,
            pltpu.SemaphoreType.DMA,
            pltpu.SemaphoreType.DMA,
        ],
        compiler_params=pltpu.CompilerParams(use_tc_tiling_on_sc=False),
    )
    def k(x_hbm, table_hbm, out_hbm, idx_v, rows0, rows1, g0, g1, s0, s1):
        cid = lax.axis_index("c")
        sid = lax.axis_index("s")
        wid = sid * NUM_CORES + cid
        base = wid * b_per_w

        pltpu.sync_copy(x_hbm.at[pl.ds(base, b_per_w)], idx_v)

        def gather(i, rows, sem):
            ic = jnp.minimum(i, n_chunks - 1)
            return pltpu.make_async_copy(
                table_hbm.at[idx_v.at[pl.ds(ic * CHUNK, CHUNK)]], rows, sem
            )

        def store(i, rows, sem):
            return pltpu.make_async_copy(
                rows, out_hbm.at[pl.ds(base + i * CHUNK, CHUNK)], sem
            )

        def scale(rows):
            def scale_row(r, carry):
                for c4 in range(D_MODEL // LANES):
                    sl = pl.ds(c4 * LANES, LANES)
                    rows[r, sl] = rows[r, sl] * SCALE
                return carry

            lax.fori_loop(0, CHUNK, scale_row, 0, unroll=4)

        gather(0, rows0, g0).start()
        gather(1, rows1, g1).start()

        def body(j, carry):
            i = j * 2
            gather(i, rows0, g0).wait()
            scale(rows0)
            store(i, rows0, s0).start()
            gather(i + 1, rows1, g1).wait()
            scale(rows1)
            store(i + 1, rows1, s1).start()
            store(i, rows0, s0).wait()
            gather(i + 2, rows0, g0).start()
            store(i + 1, rows1, s1).wait()
            gather(i + 3, rows1, g1).start()
            return carry

        lax.fori_loop(0, n_chunks // 2, body, 0)

        gather(n_chunks - 1, rows0, g0).wait()
        gather(n_chunks - 1, rows1, g1).wait()

    return k


def kernel(x, table):
    b0, b1 = x.shape
    n_rows = b0 * b1
    x_flat = x.reshape(n_rows).astype(jnp.int32)
    out = _emb_kernel(n_rows)(x_flat, table)
    return out.reshape(b0, b1, D_MODEL)
